# Initial kernel scaffold; baseline (speedup 1.0000x reference)
import jax
import jax.numpy as jnp
from jax.experimental import pallas as pl


def kernel(x, edge_index, W1, a1s, a1d, b1, W2, a2s, a2d, b2):
    raise NotImplementedError("write your pallas kernel here")


# speedup vs baseline: 53.9198x; 53.9198x over previous
import functools

import jax
import jax.numpy as jnp
from jax import lax
from jax.experimental import pallas as pl
from jax.experimental.pallas import tpu as pltpu
from jax.experimental.pallas import tpu_sc as plsc

N = 10000
D = 128
H1, C1 = 8, 8
NC = 40
E = 320000

T1W = 72
T2W = 48
CH = 128
NCHUNKS = E // CH
NCORES, NSUB = 2, 16
NTILES = NCORES * NSUB
RPS = 632
RPSL = N - (NSUB - 1) * RPS
BR = 1000
F32 = jnp.float32
I32 = jnp.int32


def _l1_dense_body(x_ref, w_ref, as_ref, ad_ref, t_ref, ad1_ref):
    h = jnp.dot(x_ref[...], w_ref[...], preferred_element_type=F32)
    asrc = jnp.dot(h, as_ref[...], preferred_element_type=F32)
    t_ref[...] = jnp.concatenate([h, asrc], axis=1)
    ad1_ref[...] = jnp.dot(h, ad_ref[...], preferred_element_type=F32)


def _l1_dense(x, W1, As, Ad):
    return pl.pallas_call(
        _l1_dense_body,
        grid=(N // BR,),
        in_specs=[
            pl.BlockSpec((BR, D), lambda i: (i, 0)),
            pl.BlockSpec((D, H1 * C1), lambda i: (0, 0)),
            pl.BlockSpec((H1 * C1, H1), lambda i: (0, 0)),
            pl.BlockSpec((H1 * C1, H1), lambda i: (0, 0)),
        ],
        out_specs=[
            pl.BlockSpec((BR, T1W), lambda i: (i, 0)),
            pl.BlockSpec((BR, H1), lambda i: (i, 0)),
        ],
        out_shape=[
            jax.ShapeDtypeStruct((N, T1W), F32),
            jax.ShapeDtypeStruct((N, H1), F32),
        ],
    )(x, W1, As, Ad)


def _edge1_body(t_hbm, ad_hbm, src_hbm, dst_hbm, z_hbm, out_hbm,
                acc_sh, src_v, dst_v, rows_v, adst_v, ex_v, msg_v, sem1, sem2):
    c = lax.axis_index("c")
    s = lax.axis_index("s")
    w = c * NSUB + s
    lane = lax.iota(I32, 16)
    half = lane >> 3
    lane8 = lane & 7

    off = pl.multiple_of(s * RPS, 8)

    @pl.when(s < NSUB - 1)
    def _zero_main():
        pltpu.sync_copy(z_hbm, acc_sh.at[pl.ds(off, RPS)])

    @pl.when(s == NSUB - 1)
    def _zero_tail():
        pltpu.sync_copy(z_hbm.at[pl.ds(0, RPSL)], acc_sh.at[pl.ds(off, RPSL)])

    plsc.subcore_barrier()

    def chunk_body(k, carry):
        base = k * CH
        pltpu.sync_copy(src_hbm.at[pl.ds(base, CH)], src_v)
        pltpu.sync_copy(dst_hbm.at[pl.ds(base, CH)], dst_v)
        cp1 = pltpu.async_copy(t_hbm.at[src_v], rows_v, sem1)
        cp2 = pltpu.async_copy(ad_hbm.at[dst_v], adst_v, sem2)
        cp1.wait()
        cp2.wait()

        def pair_body(j, cc):
            rix = 2 * j + half
            asrc = plsc.load_gather(rows_v, [rix, 64 + lane8])
            adst = plsc.load_gather(adst_v, [rix, lane8])
            al = asrc + adst
            al = jnp.where(al >= 0.0, al, 0.2 * al)
            ex = jnp.exp(al)
            plsc.store_scatter(ex_v, [rix, lane8], ex)
            plsc.store_scatter(msg_v, [rix, 64 + lane8], ex)
            return cc

        lax.fori_loop(0, CH // 2, pair_body, 0)

        def msg_body(e, cc):
            rix = jnp.full((16,), e, I32)
            for g in range(4):
                h16 = rows_v[e, pl.ds(g * 16, 16)]
                exb = plsc.load_gather(ex_v, [rix, 2 * g + half])
                msg_v[e, pl.ds(g * 16, 16)] = h16 * exb
            return cc

        lax.fori_loop(0, CH, msg_body, 0)

        pltpu.sync_copy(msg_v, acc_sh.at[dst_v], add=True)
        return carry

    lo = (w * NCHUNKS) // NTILES
    hi = ((w + 1) * NCHUNKS) // NTILES
    lax.fori_loop(lo, hi, chunk_body, 0)

    plsc.subcore_barrier()

    @pl.when(s < NSUB - 1)
    def _wb_main():
        pltpu.sync_copy(acc_sh.at[pl.ds(off, RPS)],
                        out_hbm.at[c, pl.ds(off, RPS)])

    @pl.when(s == NSUB - 1)
    def _wb_tail():
        pltpu.sync_copy(acc_sh.at[pl.ds(off, RPSL)],
                        out_hbm.at[c, pl.ds(off, RPSL)])


def _edge1(table1, adst1, src, dst, z1):
    return pl.kernel(
        _edge1_body,
        out_type=jax.ShapeDtypeStruct((NCORES, N, T1W), F32),
        mesh=plsc.VectorSubcoreMesh(core_axis_name="c", subcore_axis_name="s",
                                    num_cores=NCORES, num_subcores=NSUB),
        compiler_params=pltpu.CompilerParams(needs_layout_passes=False,
                                             use_tc_tiling_on_sc=False),
        scratch_types=[
            pltpu.VMEM_SHARED((N, T1W), F32),
            pltpu.VMEM((CH,), I32),
            pltpu.VMEM((CH,), I32),
            pltpu.VMEM((CH, T1W), F32),
            pltpu.VMEM((CH, H1), F32),
            pltpu.VMEM((CH, H1), F32),
            pltpu.VMEM((CH, T1W), F32),
            pltpu.SemaphoreType.DMA,
            pltpu.SemaphoreType.DMA,
        ],
    )(table1, adst1, src, dst, z1)


def _mid_dense_body(p_ref, b1_ref, w2_ref, a2s_ref, a2d_ref, t2_ref, ad2_ref):
    p = p_ref[0] + p_ref[1]
    den = p[:, 64:72] + 1e-16
    parts = [p[:, 8 * h:8 * h + 8] / den[:, h:h + 1] for h in range(H1)]
    hv = jnp.concatenate(parts, axis=1) + b1_ref[...]
    hv = jnp.where(hv > 0.0, hv, jnp.exp(hv) - 1.0)
    h2 = jnp.dot(hv, w2_ref[...], preferred_element_type=F32)
    asrc = jnp.sum(h2 * a2s_ref[...], axis=1, keepdims=True)
    adst = jnp.sum(h2 * a2d_ref[...], axis=1, keepdims=True)
    ones = jnp.ones((BR, 1), F32)
    pad = jnp.zeros((BR, T2W - NC - 2), F32)
    t2_ref[...] = jnp.concatenate([h2, ones, asrc, pad], axis=1)
    ad2_ref[...] = jnp.broadcast_to(adst, (BR, 8))


def _mid_dense(outp1, b1, W2, a2s, a2d):
    return pl.pallas_call(
        _mid_dense_body,
        grid=(N // BR,),
        in_specs=[
            pl.BlockSpec((NCORES, BR, T1W), lambda i: (0, i, 0)),
            pl.BlockSpec((1, H1 * C1), lambda i: (0, 0)),
            pl.BlockSpec((H1 * C1, NC), lambda i: (0, 0)),
            pl.BlockSpec((1, NC), lambda i: (0, 0)),
            pl.BlockSpec((1, NC), lambda i: (0, 0)),
        ],
        out_specs=[
            pl.BlockSpec((BR, T2W), lambda i: (i, 0)),
            pl.BlockSpec((BR, 8), lambda i: (i, 0)),
        ],
        out_shape=[
            jax.ShapeDtypeStruct((N, T2W), F32),
            jax.ShapeDtypeStruct((N, 8), F32),
        ],
    )(outp1, b1, W2, a2s, a2d)


def _edge2_body(t_hbm, ad_hbm, src_hbm, dst_hbm, z_hbm, out_hbm,
                acc_sh, src_v, dst_v, rows_v, adst_v, ex_v, msg_v, sem1, sem2):
    c = lax.axis_index("c")
    s = lax.axis_index("s")
    w = c * NSUB + s
    lane = lax.iota(I32, 16)

    off = pl.multiple_of(s * RPS, 8)

    @pl.when(s < NSUB - 1)
    def _zero_main():
        pltpu.sync_copy(z_hbm, acc_sh.at[pl.ds(off, RPS)])

    @pl.when(s == NSUB - 1)
    def _zero_tail():
        pltpu.sync_copy(z_hbm.at[pl.ds(0, RPSL)], acc_sh.at[pl.ds(off, RPSL)])

    plsc.subcore_barrier()

    def chunk_body(k, carry):
        base = k * CH
        pltpu.sync_copy(src_hbm.at[pl.ds(base, CH)], src_v)
        pltpu.sync_copy(dst_hbm.at[pl.ds(base, CH)], dst_v)
        cp1 = pltpu.async_copy(t_hbm.at[src_v], rows_v, sem1)
        cp2 = pltpu.async_copy(ad_hbm.at[dst_v], adst_v, sem2)
        cp1.wait()
        cp2.wait()

        def alpha_body(j, cc):
            rix = 16 * j + lane
            asrc = plsc.load_gather(rows_v, [rix, jnp.full((16,), NC + 1, I32)])
            adst = plsc.load_gather(adst_v, [rix, jnp.zeros((16,), I32)])
            al = asrc + adst
            al = jnp.where(al >= 0.0, al, 0.2 * al)
            ex_v[pl.ds(16 * j, 16)] = jnp.exp(al)
            return cc

        lax.fori_loop(0, CH // 16, alpha_body, 0)

        def msg_body(e, cc):
            exb = plsc.load_gather(ex_v, [jnp.full((16,), e, I32)])
            for g in range(3):
                h16 = rows_v[e, pl.ds(g * 16, 16)]
                msg_v[e, pl.ds(g * 16, 16)] = h16 * exb
            return cc

        lax.fori_loop(0, CH, msg_body, 0)

        pltpu.sync_copy(msg_v, acc_sh.at[dst_v], add=True)
        return carry

    lo = (w * NCHUNKS) // NTILES
    hi = ((w + 1) * NCHUNKS) // NTILES
    lax.fori_loop(lo, hi, chunk_body, 0)

    plsc.subcore_barrier()

    @pl.when(s < NSUB - 1)
    def _wb_main():
        pltpu.sync_copy(acc_sh.at[pl.ds(off, RPS)],
                        out_hbm.at[c, pl.ds(off, RPS)])

    @pl.when(s == NSUB - 1)
    def _wb_tail():
        pltpu.sync_copy(acc_sh.at[pl.ds(off, RPSL)],
                        out_hbm.at[c, pl.ds(off, RPSL)])


def _edge2(table2, adst2, src, dst, z2):
    return pl.kernel(
        _edge2_body,
        out_type=jax.ShapeDtypeStruct((NCORES, N, T2W), F32),
        mesh=plsc.VectorSubcoreMesh(core_axis_name="c", subcore_axis_name="s",
                                    num_cores=NCORES, num_subcores=NSUB),
        compiler_params=pltpu.CompilerParams(needs_layout_passes=False,
                                             use_tc_tiling_on_sc=False),
        scratch_types=[
            pltpu.VMEM_SHARED((N, T2W), F32),
            pltpu.VMEM((CH,), I32),
            pltpu.VMEM((CH,), I32),
            pltpu.VMEM((CH, T2W), F32),
            pltpu.VMEM((CH, 8), F32),
            pltpu.VMEM((CH,), F32),
            pltpu.VMEM((CH, T2W), F32),
            pltpu.SemaphoreType.DMA,
            pltpu.SemaphoreType.DMA,
        ],
    )(table2, adst2, src, dst, z2)


def _final_body(p_ref, b2_ref, o_ref):
    p = p_ref[0] + p_ref[1]
    o = p[:, :NC] / (p[:, NC:NC + 1] + 1e-16) + b2_ref[...]
    m = jnp.max(o, axis=1, keepdims=True)
    sh = o - m
    o_ref[...] = sh - jnp.log(jnp.sum(jnp.exp(sh), axis=1, keepdims=True))


def _final(outp2, b2):
    return pl.pallas_call(
        _final_body,
        grid=(N // BR,),
        in_specs=[
            pl.BlockSpec((NCORES, BR, T2W), lambda i: (0, i, 0)),
            pl.BlockSpec((1, NC), lambda i: (0, 0)),
        ],
        out_specs=pl.BlockSpec((BR, NC), lambda i: (i, 0)),
        out_shape=jax.ShapeDtypeStruct((N, NC), F32),
    )(outp2, b2)


def kernel(x, edge_index, W1, a1s, a1d, b1, W2, a2s, a2d, b2):
    src = edge_index[0]
    dst = edge_index[1]
    eye = jnp.eye(H1, dtype=F32)
    As = (a1s[0][:, :, None] * eye[:, None, :]).reshape(H1 * C1, H1)
    Ad = (a1d[0][:, :, None] * eye[:, None, :]).reshape(H1 * C1, H1)

    table1, adst1 = _l1_dense(x, W1, As, Ad)
    z1 = jnp.zeros((RPS, T1W), F32)
    outp1 = _edge1(table1, adst1, src, dst, z1)

    table2, adst2 = _mid_dense(outp1, b1.reshape(1, H1 * C1), W2,
                               a2s.reshape(1, NC), a2d.reshape(1, NC))
    z2 = jnp.zeros((RPS, T2W), F32)
    outp2 = _edge2(table2, adst2, src, dst, z2)

    return _final(outp2, b2.reshape(1, NC))


# speedup vs baseline: 84.5767x; 1.5686x over previous
import functools

import jax
import jax.numpy as jnp
from jax import lax
from jax.experimental import pallas as pl
from jax.experimental.pallas import tpu as pltpu
from jax.experimental.pallas import tpu_sc as plsc

N = 10000
D = 128
H1, C1 = 8, 8
NC = 40
E = 320000

T1W = 72
T2W = 48
CH = 80
NCORES, NSUB = 2, 16
NTILES = NCORES * NSUB
NCHT = E // (CH * NTILES)
RPS = 632
RPSL = N - (NSUB - 1) * RPS
BR = 1000
F32 = jnp.float32
I32 = jnp.int32

_SC_PARAMS = dict(
    compiler_params=pltpu.CompilerParams(needs_layout_passes=False,
                                         use_tc_tiling_on_sc=False),
)


def _l1_dense_body(x_ref, w_ref, as_ref, ad_ref, t_ref, ad1_ref):
    h = jnp.dot(x_ref[...], w_ref[...], preferred_element_type=F32)
    asrc = jnp.dot(h, as_ref[...], preferred_element_type=F32)
    t_ref[...] = jnp.concatenate([h, asrc], axis=1)
    ad1_ref[...] = jnp.dot(h, ad_ref[...], preferred_element_type=F32)


def _l1_dense(x, W1, As, Ad):
    return pl.pallas_call(
        _l1_dense_body,
        grid=(N // BR,),
        in_specs=[
            pl.BlockSpec((BR, D), lambda i: (i, 0)),
            pl.BlockSpec((D, H1 * C1), lambda i: (0, 0)),
            pl.BlockSpec((H1 * C1, H1), lambda i: (0, 0)),
            pl.BlockSpec((H1 * C1, H1), lambda i: (0, 0)),
        ],
        out_specs=[
            pl.BlockSpec((BR, T1W), lambda i: (i, 0)),
            pl.BlockSpec((BR, H1), lambda i: (i, 0)),
        ],
        out_shape=[
            jax.ShapeDtypeStruct((N, T1W), F32),
            jax.ShapeDtypeStruct((N, H1), F32),
        ],
    )(x, W1, As, Ad)


def _edge_body_factory(tw, adw, compute_chunk):
    def body(t_hbm, ad_hbm, src_hbm, dst_hbm, z_hbm, out_hbm,
             acc_sh, src_all, dst_all, rows0, rows1, adst0, adst1, ex_v,
             msg0, msg1, gsem0, gsem1, ssem0, ssem1):
        c = lax.axis_index("c")
        s = lax.axis_index("s")
        w = c * NSUB + s
        rows = (rows0, rows1)
        adst = (adst0, adst1)
        msg = (msg0, msg1)
        gsem = (gsem0, gsem1)
        ssem = (ssem0, ssem1)

        pltpu.sync_copy(src_hbm.at[w], src_all)
        pltpu.sync_copy(dst_hbm.at[w], dst_all)

        off = pl.multiple_of(s * RPS, 8)

        @pl.when(s < NSUB - 1)
        def _zero_main():
            pltpu.sync_copy(z_hbm, acc_sh.at[pl.ds(off, RPS)])

        @pl.when(s == NSUB - 1)
        def _zero_tail():
            pltpu.sync_copy(z_hbm.at[pl.ds(0, RPSL)], acc_sh.at[pl.ds(off, RPSL)])

        plsc.subcore_barrier()

        def issue(j, b):
            pltpu.async_copy(t_hbm.at[src_all.at[j]], rows[b], gsem[b])
            pltpu.async_copy(ad_hbm.at[dst_all.at[j]], adst[b], gsem[b])

        def wait_gather(b):
            pltpu.make_async_copy(z_hbm.at[pl.ds(0, CH)], rows[b], gsem[b]).wait()
            pltpu.make_async_copy(ad_hbm.at[pl.ds(0, CH)], adst[b], gsem[b]).wait()

        def wait_scatter(b):
            pltpu.make_async_copy(z_hbm.at[pl.ds(0, CH)], msg[b], ssem[b]).wait()

        issue(0, 0)
        issue(1, 1)

        def outer(jj, carry):
            for b in range(2):
                j = 2 * jj + b
                wait_gather(b)

                @pl.when(jj > 0)
                def _drain_prev():
                    wait_scatter(b)

                compute_chunk(rows[b], adst[b], ex_v, msg[b])
                pltpu.async_copy(msg[b], acc_sh.at[dst_all.at[j]], ssem[b],
                                 add=True)

                @pl.when(j + 2 < NCHT)
                def _prefetch():
                    issue(j + 2, b)

            return carry

        lax.fori_loop(0, NCHT // 2, outer, 0)

        wait_gather(0)
        wait_scatter(0)
        compute_chunk(rows[0], adst[0], ex_v, msg[0])
        pltpu.async_copy(msg[0], acc_sh.at[dst_all.at[NCHT - 1]], ssem[0],
                         add=True)
        wait_scatter(0)
        wait_scatter(1)

        plsc.subcore_barrier()

        @pl.when(s < NSUB - 1)
        def _wb_main():
            pltpu.sync_copy(acc_sh.at[pl.ds(off, RPS)],
                            out_hbm.at[c, pl.ds(off, RPS)])

        @pl.when(s == NSUB - 1)
        def _wb_tail():
            pltpu.sync_copy(acc_sh.at[pl.ds(off, RPSL)],
                            out_hbm.at[c, pl.ds(off, RPSL)])

    return body


def _compute1(rows_v, adst_v, ex_v, msg_v):
    lane = lax.iota(I32, 16)
    half = lane >> 3
    lane8 = lane & 7

    def pair_body(j2, cc):
        rix = 2 * j2 + half
        a = (plsc.load_gather(rows_v, [rix, 64 + lane8])
             + plsc.load_gather(adst_v, [rix, lane8]))
        a = jnp.where(a >= 0.0, a, 0.2 * a)
        ex = jnp.exp(a)
        plsc.store_scatter(ex_v, [rix, lane8], ex)
        plsc.store_scatter(msg_v, [rix, 64 + lane8], ex)
        return cc

    lax.fori_loop(0, CH // 2, pair_body, 0)

    def msg_body(e, cc):
        rix = jnp.full((16,), e, I32)
        for g in range(4):
            h16 = rows_v[e, pl.ds(g * 16, 16)]
            exb = plsc.load_gather(ex_v, [rix, 2 * g + half])
            msg_v[e, pl.ds(g * 16, 16)] = h16 * exb
        return cc

    lax.fori_loop(0, CH, msg_body, 0)


def _compute2(rows_v, adst_v, ex_v, msg_v):
    lane = lax.iota(I32, 16)
    c41 = jnp.full((16,), NC + 1, I32)
    c0 = jnp.zeros((16,), I32)

    def alpha_body(j16, cc):
        rix = 16 * j16 + lane
        a = (plsc.load_gather(rows_v, [rix, c41])
             + plsc.load_gather(adst_v, [rix, c0]))
        a = jnp.where(a >= 0.0, a, 0.2 * a)
        ex_v[pl.ds(16 * j16, 16)] = jnp.exp(a)
        return cc

    lax.fori_loop(0, CH // 16, alpha_body, 0)

    def msg_body(e, cc):
        exb = plsc.load_gather(ex_v, [jnp.full((16,), e, I32)])
        for g in range(3):
            msg_v[e, pl.ds(g * 16, 16)] = rows_v[e, pl.ds(g * 16, 16)] * exb
        return cc

    lax.fori_loop(0, CH, msg_body, 0)


def _edge_call(tw, adw, ex_shape, compute_chunk):
    return pl.kernel(
        _edge_body_factory(tw, adw, compute_chunk),
        out_type=jax.ShapeDtypeStruct((NCORES, N, tw), F32),
        mesh=plsc.VectorSubcoreMesh(core_axis_name="c", subcore_axis_name="s",
                                    num_cores=NCORES, num_subcores=NSUB),
        scratch_types=[
            pltpu.VMEM_SHARED((N, tw), F32),
            pltpu.VMEM((NCHT, CH), I32),
            pltpu.VMEM((NCHT, CH), I32),
            pltpu.VMEM((CH, tw), F32),
            pltpu.VMEM((CH, tw), F32),
            pltpu.VMEM((CH, adw), F32),
            pltpu.VMEM((CH, adw), F32),
            pltpu.VMEM(ex_shape, F32),
            pltpu.VMEM((CH, tw), F32),
            pltpu.VMEM((CH, tw), F32),
            pltpu.SemaphoreType.DMA,
            pltpu.SemaphoreType.DMA,
            pltpu.SemaphoreType.DMA,
            pltpu.SemaphoreType.DMA,
        ],
        **_SC_PARAMS,
    )


def _mid_dense_body(p_ref, b1_ref, w2_ref, a2s_ref, a2d_ref, t2_ref, ad2_ref):
    p = p_ref[0] + p_ref[1]
    den = p[:, 64:72] + 1e-16
    parts = [p[:, 8 * h:8 * h + 8] / den[:, h:h + 1] for h in range(H1)]
    hv = jnp.concatenate(parts, axis=1) + b1_ref[...]
    hv = jnp.where(hv > 0.0, hv, jnp.exp(hv) - 1.0)
    h2 = jnp.dot(hv, w2_ref[...], preferred_element_type=F32)
    asrc = jnp.sum(h2 * a2s_ref[...], axis=1, keepdims=True)
    adst = jnp.sum(h2 * a2d_ref[...], axis=1, keepdims=True)
    ones = jnp.ones((BR, 1), F32)
    pad = jnp.zeros((BR, T2W - NC - 2), F32)
    t2_ref[...] = jnp.concatenate([h2, ones, asrc, pad], axis=1)
    ad2_ref[...] = jnp.broadcast_to(adst, (BR, 8))


def _mid_dense(outp1, b1, W2, a2s, a2d):
    return pl.pallas_call(
        _mid_dense_body,
        grid=(N // BR,),
        in_specs=[
            pl.BlockSpec((NCORES, BR, T1W), lambda i: (0, i, 0)),
            pl.BlockSpec((1, H1 * C1), lambda i: (0, 0)),
            pl.BlockSpec((H1 * C1, NC), lambda i: (0, 0)),
            pl.BlockSpec((1, NC), lambda i: (0, 0)),
            pl.BlockSpec((1, NC), lambda i: (0, 0)),
        ],
        out_specs=[
            pl.BlockSpec((BR, T2W), lambda i: (i, 0)),
            pl.BlockSpec((BR, 8), lambda i: (i, 0)),
        ],
        out_shape=[
            jax.ShapeDtypeStruct((N, T2W), F32),
            jax.ShapeDtypeStruct((N, 8), F32),
        ],
    )(outp1, b1, W2, a2s, a2d)


def _final_body(p_ref, b2_ref, o_ref):
    p = p_ref[0] + p_ref[1]
    o = p[:, :NC] / (p[:, NC:NC + 1] + 1e-16) + b2_ref[...]
    m = jnp.max(o, axis=1, keepdims=True)
    sh = o - m
    o_ref[...] = sh - jnp.log(jnp.sum(jnp.exp(sh), axis=1, keepdims=True))


def _final(outp2, b2):
    return pl.pallas_call(
        _final_body,
        grid=(N // BR,),
        in_specs=[
            pl.BlockSpec((NCORES, BR, T2W), lambda i: (0, i, 0)),
            pl.BlockSpec((1, NC), lambda i: (0, 0)),
        ],
        out_specs=pl.BlockSpec((BR, NC), lambda i: (i, 0)),
        out_shape=jax.ShapeDtypeStruct((N, NC), F32),
    )(outp2, b2)


def kernel(x, edge_index, W1, a1s, a1d, b1, W2, a2s, a2d, b2):
    src = edge_index[0].reshape(NTILES, NCHT, CH)
    dst = edge_index[1].reshape(NTILES, NCHT, CH)
    eye = jnp.eye(H1, dtype=F32)
    As = (a1s[0][:, :, None] * eye[:, None, :]).reshape(H1 * C1, H1)
    Ad = (a1d[0][:, :, None] * eye[:, None, :]).reshape(H1 * C1, H1)

    table1, adst1 = _l1_dense(x, W1, As, Ad)
    z1 = jnp.zeros((RPS, T1W), F32)
    outp1 = _edge_call(T1W, H1, (CH, H1), _compute1)(
        table1, adst1, src, dst, z1)

    table2, adst2 = _mid_dense(outp1, b1.reshape(1, H1 * C1), W2,
                               a2s.reshape(1, NC), a2d.reshape(1, NC))
    z2 = jnp.zeros((RPS, T2W), F32)
    outp2 = _edge_call(T2W, 8, (CH,), _compute2)(
        table2, adst2, src, dst, z2)

    return _final(outp2, b2.reshape(1, NC))


# speedup vs baseline: 159.1356x; 1.8816x over previous
import functools

import jax
import jax.numpy as jnp
from jax import lax
from jax.experimental import pallas as pl
from jax.experimental.pallas import tpu as pltpu
from jax.experimental.pallas import tpu_sc as plsc

N = 10000
D = 128
H1, C1 = 8, 8
NC = 40
E = 320000

T1W = 72
T2W = 48
CH = 80
NCORES, NSUB = 2, 16
NTILES = NCORES * NSUB
NCHT = E // (CH * NTILES)
RPS = 632
RPSL = N - (NSUB - 1) * RPS
BR = 1000
F32 = jnp.float32
I32 = jnp.int32

_SC_PARAMS = dict(
    compiler_params=pltpu.CompilerParams(needs_layout_passes=False,
                                         use_tc_tiling_on_sc=False),
)


def _l1_dense_body(x_ref, w_ref, as_ref, ad_ref, t_ref, ad1_ref):
    h = jnp.dot(x_ref[...], w_ref[...], preferred_element_type=F32)
    asrc = jnp.dot(h, as_ref[...], preferred_element_type=F32)
    t_ref[...] = jnp.concatenate([h, asrc], axis=1)
    ad1_ref[...] = jnp.dot(h, ad_ref[...], preferred_element_type=F32)


def _l1_dense(x, W1, As, Ad):
    return pl.pallas_call(
        _l1_dense_body,
        grid=(N // BR,),
        in_specs=[
            pl.BlockSpec((BR, D), lambda i: (i, 0)),
            pl.BlockSpec((D, H1 * C1), lambda i: (0, 0)),
            pl.BlockSpec((H1 * C1, H1), lambda i: (0, 0)),
            pl.BlockSpec((H1 * C1, H1), lambda i: (0, 0)),
        ],
        out_specs=[
            pl.BlockSpec((BR, T1W), lambda i: (i, 0)),
            pl.BlockSpec((BR, H1), lambda i: (i, 0)),
        ],
        out_shape=[
            jax.ShapeDtypeStruct((N, T1W), F32),
            jax.ShapeDtypeStruct((N, H1), F32),
        ],
    )(x, W1, As, Ad)


def _edge_body_factory(tw, adw, compute_chunk):
    def body(t_hbm, ad_hbm, src_hbm, dst_hbm, z_hbm, out_hbm,
             acc_sh, src_all, dst_all, rows0, rows1, adst0, adst1, ex_v,
             msg0, msg1, gsem0, gsem1, ssem0, ssem1):
        c = lax.axis_index("c")
        s = lax.axis_index("s")
        w = c * NSUB + s
        rows = (rows0, rows1)
        adst = (adst0, adst1)
        msg = (msg0, msg1)
        gsem = (gsem0, gsem1)
        ssem = (ssem0, ssem1)

        pltpu.sync_copy(src_hbm.at[w], src_all)
        pltpu.sync_copy(dst_hbm.at[w], dst_all)

        off = pl.multiple_of(s * RPS, 8)

        @pl.when(s < NSUB - 1)
        def _zero_main():
            pltpu.sync_copy(z_hbm, acc_sh.at[pl.ds(off, RPS)])

        @pl.when(s == NSUB - 1)
        def _zero_tail():
            pltpu.sync_copy(z_hbm.at[pl.ds(0, RPSL)], acc_sh.at[pl.ds(off, RPSL)])

        plsc.subcore_barrier()

        def issue(j, b):
            pltpu.async_copy(t_hbm.at[src_all.at[j]], rows[b], gsem[b])
            pltpu.async_copy(ad_hbm.at[dst_all.at[j]], adst[b], gsem[b])

        def wait_gather(b):
            pltpu.make_async_copy(z_hbm.at[pl.ds(0, CH)], rows[b], gsem[b]).wait()
            pltpu.make_async_copy(ad_hbm.at[pl.ds(0, CH)], adst[b], gsem[b]).wait()

        def wait_scatter(b):
            pltpu.make_async_copy(z_hbm.at[pl.ds(0, CH)], msg[b], ssem[b]).wait()

        issue(0, 0)
        issue(1, 1)

        def outer(jj, carry):
            for b in range(2):
                j = 2 * jj + b
                wait_gather(b)

                @pl.when(jj > 0)
                def _drain_prev():
                    wait_scatter(b)

                compute_chunk(rows[b], adst[b], ex_v, msg[b])
                pltpu.async_copy(msg[b], acc_sh.at[dst_all.at[j]], ssem[b],
                                 add=True)

                @pl.when(j + 2 < NCHT)
                def _prefetch():
                    issue(j + 2, b)

            return carry

        lax.fori_loop(0, NCHT // 2, outer, 0)

        wait_gather(0)
        wait_scatter(0)
        compute_chunk(rows[0], adst[0], ex_v, msg[0])
        pltpu.async_copy(msg[0], acc_sh.at[dst_all.at[NCHT - 1]], ssem[0],
                         add=True)
        wait_scatter(0)
        wait_scatter(1)

        plsc.subcore_barrier()

        @pl.when(s < NSUB - 1)
        def _wb_main():
            pltpu.sync_copy(acc_sh.at[pl.ds(off, RPS)],
                            out_hbm.at[c, pl.ds(off, RPS)])

        @pl.when(s == NSUB - 1)
        def _wb_tail():
            pltpu.sync_copy(acc_sh.at[pl.ds(off, RPSL)],
                            out_hbm.at[c, pl.ds(off, RPSL)])

    return body


def _compute1(rows_v, adst_v, ex_v, msg_v):
    lane = lax.iota(I32, 16)
    half = lane >> 3
    lane8 = lane & 7

    @plsc.parallel_loop(0, CH // 2, unroll=4)
    def pair_body(j2):
        rix = 2 * j2 + half
        a = (plsc.load_gather(rows_v, [rix, 64 + lane8])
             + plsc.load_gather(adst_v, [rix, lane8]))
        a = jnp.where(a >= 0.0, a, 0.2 * a)
        ex = jnp.exp(a)
        plsc.store_scatter(ex_v, [rix, lane8], ex)
        plsc.store_scatter(msg_v, [rix, 64 + lane8], ex)

    @plsc.parallel_loop(0, CH, unroll=2)
    def msg_body(e):
        rix = jnp.full((16,), e, I32)
        for g in range(4):
            h16 = rows_v[e, pl.ds(g * 16, 16)]
            exb = plsc.load_gather(ex_v, [rix, 2 * g + half])
            msg_v[e, pl.ds(g * 16, 16)] = h16 * exb


def _compute2(rows_v, adst_v, ex_v, msg_v):
    lane = lax.iota(I32, 16)
    c41 = jnp.full((16,), NC + 1, I32)
    c0 = jnp.zeros((16,), I32)

    @plsc.parallel_loop(0, CH // 16, unroll=1)
    def alpha_body(j16):
        rix = 16 * j16 + lane
        a = (plsc.load_gather(rows_v, [rix, c41])
             + plsc.load_gather(adst_v, [rix, c0]))
        a = jnp.where(a >= 0.0, a, 0.2 * a)
        ex_v[pl.ds(16 * j16, 16)] = jnp.exp(a)

    @plsc.parallel_loop(0, CH, unroll=2)
    def msg_body(e):
        exb = plsc.load_gather(ex_v, [jnp.full((16,), e, I32)])
        for g in range(3):
            msg_v[e, pl.ds(g * 16, 16)] = rows_v[e, pl.ds(g * 16, 16)] * exb


def _edge_call(tw, adw, ex_shape, compute_chunk):
    return pl.kernel(
        _edge_body_factory(tw, adw, compute_chunk),
        out_type=jax.ShapeDtypeStruct((NCORES, N, tw), F32),
        mesh=plsc.VectorSubcoreMesh(core_axis_name="c", subcore_axis_name="s",
                                    num_cores=NCORES, num_subcores=NSUB),
        scratch_types=[
            pltpu.VMEM_SHARED((N, tw), F32),
            pltpu.VMEM((NCHT, CH), I32),
            pltpu.VMEM((NCHT, CH), I32),
            pltpu.VMEM((CH, tw), F32),
            pltpu.VMEM((CH, tw), F32),
            pltpu.VMEM((CH, adw), F32),
            pltpu.VMEM((CH, adw), F32),
            pltpu.VMEM(ex_shape, F32),
            pltpu.VMEM((CH, tw), F32),
            pltpu.VMEM((CH, tw), F32),
            pltpu.SemaphoreType.DMA,
            pltpu.SemaphoreType.DMA,
            pltpu.SemaphoreType.DMA,
            pltpu.SemaphoreType.DMA,
        ],
        **_SC_PARAMS,
    )


def _mid_dense_body(p_ref, b1_ref, w2_ref, a2s_ref, a2d_ref, t2_ref, ad2_ref):
    p = p_ref[0] + p_ref[1]
    den = p[:, 64:72] + 1e-16
    parts = [p[:, 8 * h:8 * h + 8] / den[:, h:h + 1] for h in range(H1)]
    hv = jnp.concatenate(parts, axis=1) + b1_ref[...]
    hv = jnp.where(hv > 0.0, hv, jnp.exp(hv) - 1.0)
    h2 = jnp.dot(hv, w2_ref[...], preferred_element_type=F32)
    asrc = jnp.sum(h2 * a2s_ref[...], axis=1, keepdims=True)
    adst = jnp.sum(h2 * a2d_ref[...], axis=1, keepdims=True)
    ones = jnp.ones((BR, 1), F32)
    pad = jnp.zeros((BR, T2W - NC - 2), F32)
    t2_ref[...] = jnp.concatenate([h2, ones, asrc, pad], axis=1)
    ad2_ref[...] = jnp.broadcast_to(adst, (BR, 8))


def _mid_dense(outp1, b1, W2, a2s, a2d):
    return pl.pallas_call(
        _mid_dense_body,
        grid=(N // BR,),
        in_specs=[
            pl.BlockSpec((NCORES, BR, T1W), lambda i: (0, i, 0)),
            pl.BlockSpec((1, H1 * C1), lambda i: (0, 0)),
            pl.BlockSpec((H1 * C1, NC), lambda i: (0, 0)),
            pl.BlockSpec((1, NC), lambda i: (0, 0)),
            pl.BlockSpec((1, NC), lambda i: (0, 0)),
        ],
        out_specs=[
            pl.BlockSpec((BR, T2W), lambda i: (i, 0)),
            pl.BlockSpec((BR, 8), lambda i: (i, 0)),
        ],
        out_shape=[
            jax.ShapeDtypeStruct((N, T2W), F32),
            jax.ShapeDtypeStruct((N, 8), F32),
        ],
    )(outp1, b1, W2, a2s, a2d)


def _final_body(p_ref, b2_ref, o_ref):
    p = p_ref[0] + p_ref[1]
    o = p[:, :NC] / (p[:, NC:NC + 1] + 1e-16) + b2_ref[...]
    m = jnp.max(o, axis=1, keepdims=True)
    sh = o - m
    o_ref[...] = sh - jnp.log(jnp.sum(jnp.exp(sh), axis=1, keepdims=True))


def _final(outp2, b2):
    return pl.pallas_call(
        _final_body,
        grid=(N // BR,),
        in_specs=[
            pl.BlockSpec((NCORES, BR, T2W), lambda i: (0, i, 0)),
            pl.BlockSpec((1, NC), lambda i: (0, 0)),
        ],
        out_specs=pl.BlockSpec((BR, NC), lambda i: (i, 0)),
        out_shape=jax.ShapeDtypeStruct((N, NC), F32),
    )(outp2, b2)


def kernel(x, edge_index, W1, a1s, a1d, b1, W2, a2s, a2d, b2):
    src = edge_index[0].reshape(NTILES, NCHT, CH)
    dst = edge_index[1].reshape(NTILES, NCHT, CH)
    eye = jnp.eye(H1, dtype=F32)
    As = (a1s[0][:, :, None] * eye[:, None, :]).reshape(H1 * C1, H1)
    Ad = (a1d[0][:, :, None] * eye[:, None, :]).reshape(H1 * C1, H1)

    table1, adst1 = _l1_dense(x, W1, As, Ad)
    z1 = jnp.zeros((RPS, T1W), F32)
    outp1 = _edge_call(T1W, H1, (CH, H1), _compute1)(
        table1, adst1, src, dst, z1)

    table2, adst2 = _mid_dense(outp1, b1.reshape(1, H1 * C1), W2,
                               a2s.reshape(1, NC), a2d.reshape(1, NC))
    z2 = jnp.zeros((RPS, T2W), F32)
    outp2 = _edge_call(T2W, 8, (CH,), _compute2)(
        table2, adst2, src, dst, z2)

    return _final(outp2, b2.reshape(1, NC))


# speedup vs baseline: 193.0197x; 1.2129x over previous
import functools

import jax
import jax.numpy as jnp
from jax import lax
from jax.experimental import pallas as pl
from jax.experimental.pallas import tpu as pltpu
from jax.experimental.pallas import tpu_sc as plsc

N = 10000
D = 128
H1, C1 = 8, 8
NC = 40
E = 320000

T1W = 72
T2W = 48
CH = 100
NBUF = 4
NCORES, NSUB = 2, 16
NTILES = NCORES * NSUB
NCHT = E // (CH * NTILES)
RPS = 632
RPSL = N - (NSUB - 1) * RPS
BR = 1000
F32 = jnp.float32
I32 = jnp.int32

_SC_PARAMS = dict(
    compiler_params=pltpu.CompilerParams(needs_layout_passes=False,
                                         use_tc_tiling_on_sc=False),
)


def _l1_dense_body(x_ref, w_ref, as_ref, ad_ref, t_ref, ad1_ref):
    h = jnp.dot(x_ref[...], w_ref[...], preferred_element_type=F32)
    asrc = jnp.dot(h, as_ref[...], preferred_element_type=F32)
    t_ref[...] = jnp.concatenate([h, asrc], axis=1)
    ad1_ref[...] = jnp.dot(h, ad_ref[...], preferred_element_type=F32)


def _l1_dense(x, W1, As, Ad):
    return pl.pallas_call(
        _l1_dense_body,
        grid=(N // BR,),
        in_specs=[
            pl.BlockSpec((BR, D), lambda i: (i, 0)),
            pl.BlockSpec((D, H1 * C1), lambda i: (0, 0)),
            pl.BlockSpec((H1 * C1, H1), lambda i: (0, 0)),
            pl.BlockSpec((H1 * C1, H1), lambda i: (0, 0)),
        ],
        out_specs=[
            pl.BlockSpec((BR, T1W), lambda i: (i, 0)),
            pl.BlockSpec((BR, H1), lambda i: (i, 0)),
        ],
        out_shape=[
            jax.ShapeDtypeStruct((N, T1W), F32),
            jax.ShapeDtypeStruct((N, H1), F32),
        ],
    )(x, W1, As, Ad)


def _edge_body_factory(tw, adw, compute_chunk):
    def body(t_hbm, ad_hbm, src_hbm, dst_hbm, z_hbm, out_hbm,
             acc_sh, src_all, dst_all, *bufs):
        rows = bufs[0:NBUF]
        adst = bufs[NBUF:2 * NBUF]
        msg = bufs[2 * NBUF:3 * NBUF]
        ex_v = bufs[3 * NBUF]
        gsem = bufs[3 * NBUF + 1:3 * NBUF + 1 + NBUF]
        ssem = bufs[3 * NBUF + 1 + NBUF:]
        c = lax.axis_index("c")
        s = lax.axis_index("s")
        w = c * NSUB + s

        pltpu.sync_copy(src_hbm.at[w], src_all)
        pltpu.sync_copy(dst_hbm.at[w], dst_all)

        off = pl.multiple_of(s * RPS, 8)

        @pl.when(s < NSUB - 1)
        def _zero_main():
            pltpu.sync_copy(z_hbm, acc_sh.at[pl.ds(off, RPS)])

        @pl.when(s == NSUB - 1)
        def _zero_tail():
            pltpu.sync_copy(z_hbm.at[pl.ds(0, RPSL)], acc_sh.at[pl.ds(off, RPSL)])

        plsc.subcore_barrier()

        def issue(j, b):
            pltpu.async_copy(t_hbm.at[src_all.at[j]], rows[b], gsem[b])
            pltpu.async_copy(ad_hbm.at[dst_all.at[j]], adst[b], gsem[b])

        def wait_gather(b):
            pltpu.make_async_copy(z_hbm.at[pl.ds(0, CH)], rows[b], gsem[b]).wait()
            pltpu.make_async_copy(ad_hbm.at[pl.ds(0, CH)], adst[b], gsem[b]).wait()

        def wait_scatter(b):
            pltpu.make_async_copy(z_hbm.at[pl.ds(0, CH)], msg[b], ssem[b]).wait()

        for b in range(NBUF):
            issue(b, b)

        def outer(jj, carry):
            for b in range(NBUF):
                j = NBUF * jj + b
                wait_gather(b)

                @pl.when(jj > 0)
                def _drain_prev():
                    wait_scatter(b)

                compute_chunk(rows[b], adst[b], ex_v, msg[b])
                pltpu.async_copy(msg[b], acc_sh.at[dst_all.at[j]], ssem[b],
                                 add=True)

                @pl.when(j + NBUF < NCHT)
                def _prefetch():
                    issue(j + NBUF, b)

            return carry

        lax.fori_loop(0, NCHT // NBUF, outer, 0)

        for b in range(NBUF):
            wait_scatter(b)

        plsc.subcore_barrier()

        @pl.when(s < NSUB - 1)
        def _wb_main():
            pltpu.sync_copy(acc_sh.at[pl.ds(off, RPS)],
                            out_hbm.at[c, pl.ds(off, RPS)])

        @pl.when(s == NSUB - 1)
        def _wb_tail():
            pltpu.sync_copy(acc_sh.at[pl.ds(off, RPSL)],
                            out_hbm.at[c, pl.ds(off, RPSL)])

    return body


def _compute1(rows_v, adst_v, ex_v, msg_v):
    lane = lax.iota(I32, 16)
    half = lane >> 3
    lane8 = lane & 7

    @plsc.parallel_loop(0, CH // 2, unroll=4)
    def pair_body(j2):
        rix = 2 * j2 + half
        a = (plsc.load_gather(rows_v, [rix, 64 + lane8])
             + plsc.load_gather(adst_v, [rix, lane8]))
        a = jnp.where(a >= 0.0, a, 0.2 * a)
        ex = jnp.exp(a)
        plsc.store_scatter(ex_v, [rix, lane8], ex)
        plsc.store_scatter(msg_v, [rix, 64 + lane8], ex)

    @plsc.parallel_loop(0, CH, unroll=2)
    def msg_body(e):
        rix = jnp.full((16,), e, I32)
        for g in range(4):
            h16 = rows_v[e, pl.ds(g * 16, 16)]
            exb = plsc.load_gather(ex_v, [rix, 2 * g + half])
            msg_v[e, pl.ds(g * 16, 16)] = h16 * exb


def _compute2(rows_v, adst_v, ex_v, msg_v):
    lane = lax.iota(I32, 16)
    c41 = jnp.full((16,), NC + 1, I32)
    c0 = jnp.zeros((16,), I32)

    @plsc.parallel_loop(0, (CH + 15) // 16, unroll=1)
    def alpha_body(j16):
        rix = jnp.minimum(16 * j16 + lane, CH - 1)
        a = (plsc.load_gather(rows_v, [rix, c41])
             + plsc.load_gather(adst_v, [rix, c0]))
        a = jnp.where(a >= 0.0, a, 0.2 * a)
        ex_v[pl.ds(16 * j16, 16)] = jnp.exp(a)

    @plsc.parallel_loop(0, CH, unroll=2)
    def msg_body(e):
        exb = plsc.load_gather(ex_v, [jnp.full((16,), e, I32)])
        for g in range(3):
            msg_v[e, pl.ds(g * 16, 16)] = rows_v[e, pl.ds(g * 16, 16)] * exb


def _edge_call(tw, adw, ex_shape, compute_chunk):
    return pl.kernel(
        _edge_body_factory(tw, adw, compute_chunk),
        out_type=jax.ShapeDtypeStruct((NCORES, N, tw), F32),
        mesh=plsc.VectorSubcoreMesh(core_axis_name="c", subcore_axis_name="s",
                                    num_cores=NCORES, num_subcores=NSUB),
        scratch_types=(
            [pltpu.VMEM_SHARED((N, tw), F32)]
            + [pltpu.VMEM((NCHT, CH), I32)] * 2
            + [pltpu.VMEM((CH, tw), F32)] * NBUF
            + [pltpu.VMEM((CH, adw), F32)] * NBUF
            + [pltpu.VMEM((CH, tw), F32)] * NBUF
            + [pltpu.VMEM(ex_shape, F32)]
            + [pltpu.SemaphoreType.DMA] * (2 * NBUF)
        ),
        **_SC_PARAMS,
    )


def _mid_dense_body(p_ref, b1_ref, w2_ref, a2s_ref, a2d_ref, t2_ref, ad2_ref):
    p = p_ref[0] + p_ref[1]
    den = p[:, 64:72] + 1e-16
    parts = [p[:, 8 * h:8 * h + 8] / den[:, h:h + 1] for h in range(H1)]
    hv = jnp.concatenate(parts, axis=1) + b1_ref[...]
    hv = jnp.where(hv > 0.0, hv, jnp.exp(hv) - 1.0)
    h2 = jnp.dot(hv, w2_ref[...], preferred_element_type=F32)
    asrc = jnp.sum(h2 * a2s_ref[...], axis=1, keepdims=True)
    adst = jnp.sum(h2 * a2d_ref[...], axis=1, keepdims=True)
    ones = jnp.ones((BR, 1), F32)
    pad = jnp.zeros((BR, T2W - NC - 2), F32)
    t2_ref[...] = jnp.concatenate([h2, ones, asrc, pad], axis=1)
    ad2_ref[...] = jnp.broadcast_to(adst, (BR, 8))


def _mid_dense(outp1, b1, W2, a2s, a2d):
    return pl.pallas_call(
        _mid_dense_body,
        grid=(N // BR,),
        in_specs=[
            pl.BlockSpec((NCORES, BR, T1W), lambda i: (0, i, 0)),
            pl.BlockSpec((1, H1 * C1), lambda i: (0, 0)),
            pl.BlockSpec((H1 * C1, NC), lambda i: (0, 0)),
            pl.BlockSpec((1, NC), lambda i: (0, 0)),
            pl.BlockSpec((1, NC), lambda i: (0, 0)),
        ],
        out_specs=[
            pl.BlockSpec((BR, T2W), lambda i: (i, 0)),
            pl.BlockSpec((BR, 8), lambda i: (i, 0)),
        ],
        out_shape=[
            jax.ShapeDtypeStruct((N, T2W), F32),
            jax.ShapeDtypeStruct((N, 8), F32),
        ],
    )(outp1, b1, W2, a2s, a2d)


def _final_body(p_ref, b2_ref, o_ref):
    p = p_ref[0] + p_ref[1]
    o = p[:, :NC] / (p[:, NC:NC + 1] + 1e-16) + b2_ref[...]
    m = jnp.max(o, axis=1, keepdims=True)
    sh = o - m
    o_ref[...] = sh - jnp.log(jnp.sum(jnp.exp(sh), axis=1, keepdims=True))


def _final(outp2, b2):
    return pl.pallas_call(
        _final_body,
        grid=(N // BR,),
        in_specs=[
            pl.BlockSpec((NCORES, BR, T2W), lambda i: (0, i, 0)),
            pl.BlockSpec((1, NC), lambda i: (0, 0)),
        ],
        out_specs=pl.BlockSpec((BR, NC), lambda i: (i, 0)),
        out_shape=jax.ShapeDtypeStruct((N, NC), F32),
    )(outp2, b2)


def kernel(x, edge_index, W1, a1s, a1d, b1, W2, a2s, a2d, b2):
    src = edge_index[0].reshape(NTILES, NCHT, CH)
    dst = edge_index[1].reshape(NTILES, NCHT, CH)
    eye = jnp.eye(H1, dtype=F32)
    As = (a1s[0][:, :, None] * eye[:, None, :]).reshape(H1 * C1, H1)
    Ad = (a1d[0][:, :, None] * eye[:, None, :]).reshape(H1 * C1, H1)

    table1, adst1 = _l1_dense(x, W1, As, Ad)
    z1 = jnp.zeros((RPS, T1W), F32)
    outp1 = _edge_call(T1W, H1, (CH, H1), _compute1)(
        table1, adst1, src, dst, z1)

    table2, adst2 = _mid_dense(outp1, b1.reshape(1, H1 * C1), W2,
                               a2s.reshape(1, NC), a2d.reshape(1, NC))
    z2 = jnp.zeros((RPS, T2W), F32)
    outp2 = _edge_call(T2W, 8, (((CH + 15) // 16) * 16,), _compute2)(
        table2, adst2, src, dst, z2)

    return _final(outp2, b2.reshape(1, NC))
